# baseline (speedup 1.0000x reference)
import functools

import jax
import jax.numpy as jnp
from jax import lax
from jax.experimental import pallas as pl
from jax.experimental.pallas import tpu as pltpu
from jax.experimental.pallas import tpu_sc as plsc

D_MODEL = 256
T_ROWS = 256
NUM_CORES = 2
NUM_SUBCORES = 16
NUM_WORKERS = NUM_CORES * NUM_SUBCORES
CHUNK = 128
NBUF = 2


def _prep_kernel(card_ref, stg_ref, rank_ref, suit_ref, stage_ref,
                 t_ref, idx_ref):
    rows = lax.broadcasted_iota(jnp.int32, (T_ROWS, 1), 0)
    card = rows // 4
    stg = rows % 4
    rank = card % 13
    suit = card // 13
    valid = card < 52
    acc = jnp.zeros((T_ROWS, D_MODEL), jnp.float32)
    for k in range(13):
        acc += jnp.where(rank == k, 1.0, 0.0) * rank_ref[k, :][None, :]
    for k in range(4):
        acc += jnp.where(suit == k, 1.0, 0.0) * suit_ref[k, :][None, :]
        acc += jnp.where(stg == k, 1.0, 0.0) * stage_ref[k, :][None, :]
    t_ref[...] = jnp.where(valid, acc, 0.0)

    c = card_ref[...]
    s = stg_ref[...]
    cvalid = (c >= 0) & (c < 52)
    cc = jnp.where(cvalid, c, 52)
    ss = jnp.clip(s, 0, 3)
    idx_ref[...] = cc * 4 + ss


def _prep(card_indices, stages, rank_emb, suit_emb, stage_emb):
    batch, seq = card_indices.shape
    return pl.pallas_call(
        _prep_kernel,
        out_shape=(
            jax.ShapeDtypeStruct((T_ROWS, D_MODEL), jnp.float32),
            jax.ShapeDtypeStruct((batch, seq), jnp.int32),
        ),
    )(card_indices, stages, rank_emb, suit_emb, stage_emb)


def _make_sc_gather(n_pos):
    assert n_pos % (NUM_WORKERS * CHUNK * NBUF) == 0
    per_worker = n_pos // NUM_WORKERS
    n_chunks = per_worker // CHUNK
    mesh = plsc.VectorSubcoreMesh(core_axis_name="c", subcore_axis_name="s")

    scratch = []
    for _ in range(NBUF):
        scratch += [
            pltpu.VMEM((CHUNK,), jnp.int32),
            pltpu.VMEM((CHUNK, D_MODEL), jnp.float32),
            pltpu.SemaphoreType.DMA,
            pltpu.SemaphoreType.DMA,
            pltpu.SemaphoreType.DMA,
        ]

    @functools.partial(
        pl.kernel,
        out_type=jax.ShapeDtypeStruct((n_pos, D_MODEL), jnp.float32),
        mesh=mesh,
        scratch_types=scratch,
    )
    def sc_gather(idx_hbm, t_hbm, out_hbm, *bufs):
        idx_v = [bufs[5 * b + 0] for b in range(NBUF)]
        rows_v = [bufs[5 * b + 1] for b in range(NBUF)]
        isem = [bufs[5 * b + 2] for b in range(NBUF)]
        gsem = [bufs[5 * b + 3] for b in range(NBUF)]
        osem = [bufs[5 * b + 4] for b in range(NBUF)]
        wid = lax.axis_index("s") * NUM_CORES + lax.axis_index("c")
        base = wid * per_worker
        row0 = wid * (per_worker // CHUNK)

        def fire_in(row, b):
            pltpu.async_copy(idx_hbm.at[row], idx_v[b], isem[b])

        for b in range(NBUF):
            fire_in(row0 + b, b)

        def group(g, carry):
            for b in range(NBUF):
                c = g * NBUF + b
                off = base + c * CHUNK
                pltpu.make_async_copy(
                    idx_hbm.at[row0], idx_v[b], isem[b]).wait()

                @pl.when(g > 0)
                def _wait_prev_scatter():
                    pltpu.make_async_copy(
                        rows_v[b], out_hbm.at[pl.ds(base, CHUNK)], osem[b]).wait()

                gath = pltpu.async_copy(t_hbm.at[idx_v[b]], rows_v[b], gsem[b])

                @pl.when(c + NBUF < n_chunks)
                def _fire_next_in():
                    fire_in(row0 + c + NBUF, b)

                gath.wait()
                pltpu.async_copy(rows_v[b], out_hbm.at[pl.ds(off, CHUNK)], osem[b])
            return carry

        lax.fori_loop(0, n_chunks // NBUF, group, 0)
        for b in range(NBUF):
            pltpu.make_async_copy(
                rows_v[b], out_hbm.at[pl.ds(base, CHUNK)], osem[b]).wait()

    return sc_gather


def kernel(card_indices, stages, rank_emb, suit_emb, stage_emb):
    batch, seq = card_indices.shape
    n_pos = batch * seq
    table, idx = _prep(card_indices.astype(jnp.int32), stages.astype(jnp.int32),
                       rank_emb, suit_emb, stage_emb)
    idx2 = idx.reshape(n_pos // CHUNK, CHUNK)
    out = _make_sc_gather(n_pos)(idx2, table)
    return out.reshape(batch, seq, D_MODEL)
